# baseline (speedup 1.0000x reference)
import numpy as np
import jax
import jax.numpy as jnp
from jax import lax
from jax.experimental import pallas as pl
from jax.experimental.pallas import tpu as pltpu
from jax.experimental.pallas import tpu_sc as plsc

B = 4
S = 8192
D = 1024
PAD = 0

NC = 2
NS = 16
NW = NC * NS
POS_PER_W = S // NW
P = 8
N_CHUNKS = POS_PER_W // P
SSTR = 32


def _pos_encoding(seq_len, d_model):
    pos = np.arange(seq_len, dtype=np.float32)[:, None]
    i = np.arange(0, d_model, 2, dtype=np.float32)
    div = np.exp(-np.log(10000.0) * i / float(d_model))
    pe = np.zeros((seq_len, d_model), dtype=np.float32)
    pe[:, 0::2] = np.sin(pos * div)
    pe[:, 1::2] = np.cos(pos * div)
    u = pe.view(np.uint32)
    bf = ((u + 0x7FFF + ((u >> 16) & 1)) >> 16).astype(np.uint32)
    z = bf.reshape(seq_len, d_model // 32, 2, 16)
    w = z[:, :, 0, :] | (z[:, :, 1, :] << 16)
    return jnp.asarray(w.reshape(seq_len * d_model // 2).view(np.int32))


def _body(x_hbm, pe_hbm, tbl_hbm, out_hbm, idx_all, scale_v,
          pe0, pe1, rb0, rb1, rb2, rb3,
          g0, g1, g2, g3, s0, s1, s2, s3, q0, q1):
    wid = lax.axis_index("s") * NC + lax.axis_index("c")
    pos0 = wid * POS_PER_W
    peb = (pe0, pe1)
    rbb = (rb0, rb1, rb2, rb3)
    gs = (g0, g1, g2, g3)
    ss = (s0, s1, s2, s3)
    qs = (q0, q1)

    pltpu.sync_copy(x_hbm.at[:, pl.ds(pos0, POS_PER_W)],
                    idx_all.at[:, pl.ds(0, POS_PER_W)])

    def issue_gathers(c, h, j):
        pltpu.async_copy(
            tbl_hbm.at[idx_all.at[2 * h, pl.ds(c * P, P)]],
            rbb[j].at[0], gs[j])
        pltpu.async_copy(
            tbl_hbm.at[idx_all.at[2 * h + 1, pl.ds(c * P, P)]],
            rbb[j].at[1], gs[j])

    HD = D // 2
    pltpu.async_copy(pe_hbm.at[pl.ds(pos0 * HD, P * HD)], pe0, q0)
    pltpu.async_copy(pe_hbm.at[pl.ds((pos0 + P) * HD, P * HD)], pe1, q1)
    issue_gathers(0, 0, 0)

    def cpair_body(cp, _):
        for cc in range(2):
            c = cp * 2 + cc
            coff = c * P
            pstart = pos0 + coff
            for h in range(2):
                j = (2 * cc + h) % 4
                j1 = (j + 1) % 4
                h1 = 1 - h
                cm3 = c - 2 + h
                c1 = c + h

                @pl.when(cm3 >= 0)
                def _():
                    pltpu.make_async_copy(
                        rbb[j1].at[0],
                        out_hbm.at[2 * h1, pl.ds(pos0 + cm3 * P, P)],
                        ss[j1]).wait()
                    pltpu.make_async_copy(
                        rbb[j1].at[1],
                        out_hbm.at[2 * h1 + 1, pl.ds(pos0 + cm3 * P, P)],
                        ss[j1]).wait()

                @pl.when(c1 < N_CHUNKS)
                def _():
                    issue_gathers(c1, h1, j1)

                pltpu.make_async_copy(
                    tbl_hbm.at[idx_all.at[2 * h, pl.ds(coff, P)]],
                    rbb[j].at[0], gs[j]).wait()
                pltpu.make_async_copy(
                    tbl_hbm.at[idx_all.at[2 * h + 1, pl.ds(coff, P)]],
                    rbb[j].at[1], gs[j]).wait()

                if h == 0:
                    pltpu.make_async_copy(
                        pe_hbm.at[pl.ds(pstart * HD, P * HD)], peb[cc],
                        qs[cc]).wait()

                aoff = pl.multiple_of(cp * 16, 16)
                for bl in range(2):
                    ivec = idx_all[2 * h + bl, pl.ds(aoff, 16)]
                    scale_v[pl.ds(bl * SSTR, 16)] = jnp.where(
                        ivec != PAD, 1.0, 0.0).astype(jnp.float32)

                def row_body(r, _):
                    sc0 = scale_v[pl.ds(8 * cc + r, 16)][0]
                    sc1 = scale_v[pl.ds(SSTR + 8 * cc + r, 16)][0]

                    @plsc.parallel_loop(0, D, 32, unroll=4)
                    def _add(off):
                        poff = pl.multiple_of(r * HD + off // 2, 16)
                        w = peb[cc][pl.ds(poff, 16)]
                        pa = lax.bitcast_convert_type(
                            w << 16, jnp.float32)
                        pb = lax.bitcast_convert_type(
                            w & jnp.int32(-65536), jnp.float32)
                        off2 = off + 16
                        rbb[j][0, r, pl.ds(off, 16)] = (
                            rbb[j][0, r, pl.ds(off, 16)] * sc0 + pa)
                        rbb[j][1, r, pl.ds(off, 16)] = (
                            rbb[j][1, r, pl.ds(off, 16)] * sc1 + pa)
                        rbb[j][0, r, pl.ds(off2, 16)] = (
                            rbb[j][0, r, pl.ds(off2, 16)] * sc0 + pb)
                        rbb[j][1, r, pl.ds(off2, 16)] = (
                            rbb[j][1, r, pl.ds(off2, 16)] * sc1 + pb)
                    return 0

                lax.fori_loop(0, P, row_body, 0)

                pltpu.async_copy(
                    rbb[j].at[0], out_hbm.at[2 * h, pl.ds(pstart, P)], ss[j])
                pltpu.async_copy(
                    rbb[j].at[1], out_hbm.at[2 * h + 1, pl.ds(pstart, P)],
                    ss[j])

                if h == 1:
                    @pl.when(c + 2 < N_CHUNKS)
                    def _():
                        pltpu.async_copy(
                            pe_hbm.at[pl.ds((pstart + 2 * P) * HD, P * HD)],
                            peb[cc], qs[cc])
        return 0

    lax.fori_loop(0, N_CHUNKS // 2, cpair_body, 0)
    last = pos0 + (N_CHUNKS - 1) * P
    prev = pos0 + (N_CHUNKS - 2) * P
    pltpu.make_async_copy(rb1.at[0], out_hbm.at[2, pl.ds(prev, P)], s1).wait()
    pltpu.make_async_copy(rb1.at[1], out_hbm.at[3, pl.ds(prev, P)], s1).wait()
    pltpu.make_async_copy(rb2.at[0], out_hbm.at[0, pl.ds(last, P)], s2).wait()
    pltpu.make_async_copy(rb2.at[1], out_hbm.at[1, pl.ds(last, P)], s2).wait()
    pltpu.make_async_copy(rb3.at[0], out_hbm.at[2, pl.ds(last, P)], s3).wait()
    pltpu.make_async_copy(rb3.at[1], out_hbm.at[3, pl.ds(last, P)], s3).wait()


def kernel(x, token_emb_weight):
    pe = _pos_encoding(S, D)
    mesh = plsc.VectorSubcoreMesh(core_axis_name="c", subcore_axis_name="s")
    k = pl.kernel(
        _body,
        out_type=jax.ShapeDtypeStruct((B, S, D), jnp.float32),
        mesh=mesh,
        scratch_types=[
            pltpu.VMEM((B, POS_PER_W + 16), jnp.int32),
            pltpu.VMEM((2 * SSTR,), jnp.float32),
            pltpu.VMEM((P * D // 2,), jnp.int32),
            pltpu.VMEM((P * D // 2,), jnp.int32),
            pltpu.VMEM((2, P, D), jnp.float32),
            pltpu.VMEM((2, P, D), jnp.float32),
            pltpu.VMEM((2, P, D), jnp.float32),
            pltpu.VMEM((2, P, D), jnp.float32),
        ] + [pltpu.SemaphoreType.DMA] * 10,
    )
    return k(x, pe, token_emb_weight)


# speedup vs baseline: 1.0313x; 1.0313x over previous
import numpy as np
import jax
import jax.numpy as jnp
from jax import lax
from jax.experimental import pallas as pl
from jax.experimental.pallas import tpu as pltpu
from jax.experimental.pallas import tpu_sc as plsc

B = 4
S = 8192
D = 1024
PAD = 0

NC = 2
NS = 16
NW = NC * NS
POS_PER_W = S // NW
P = 8
N_CHUNKS = POS_PER_W // P
SSTR = 32


def _pos_encoding(seq_len, d_model):
    pos = np.arange(seq_len, dtype=np.float32)[:, None]
    i = np.arange(0, d_model, 2, dtype=np.float32)
    div = np.exp(-np.log(10000.0) * i / float(d_model))
    pe = np.zeros((seq_len, d_model), dtype=np.float32)
    pe[:, 0::2] = np.sin(pos * div)
    pe[:, 1::2] = np.cos(pos * div)
    u = pe.view(np.uint32)
    bf = ((u + 0x7FFF + ((u >> 16) & 1)) >> 16).astype(np.uint32)
    z = bf.reshape(seq_len, d_model // 32, 2, 16)
    w = z[:, :, 0, :] | (z[:, :, 1, :] << 16)
    return jnp.asarray(w.reshape(seq_len * d_model // 2).view(np.int32))


def _body(x_hbm, pe_hbm, tbl_hbm, out_hbm, idx_all, scale_v,
          pe0, pe1, rb0, rb1, rb2, rb3,
          g0, g1, g2, g3, s0, s1, s2, s3, q0, q1):
    wid = lax.axis_index("s") * NC + lax.axis_index("c")
    pos0 = wid * POS_PER_W
    peb = (pe0, pe1)
    rbb = (rb0, rb1, rb2, rb3)
    gs = (g0, g1, g2, g3)
    ss = (s0, s1, s2, s3)
    qs = (q0, q1)

    pltpu.sync_copy(x_hbm.at[:, pl.ds(pos0, POS_PER_W)],
                    idx_all.at[:, pl.ds(0, POS_PER_W)])

    def issue_gathers(c, h, j):
        pltpu.async_copy(
            tbl_hbm.at[idx_all.at[2 * h, pl.ds(c * P, P)]],
            rbb[j].at[0], gs[j])
        pltpu.async_copy(
            tbl_hbm.at[idx_all.at[2 * h + 1, pl.ds(c * P, P)]],
            rbb[j].at[1], gs[j])

    HD = D // 2
    pltpu.async_copy(pe_hbm.at[pl.ds(pos0 * HD, P * HD)], pe0, q0)
    pltpu.async_copy(pe_hbm.at[pl.ds((pos0 + P) * HD, P * HD)], pe1, q1)
    issue_gathers(0, 0, 0)
    issue_gathers(0, 1, 1)

    def cpair_body(cp, _):
        for cc in range(2):
            c = cp * 2 + cc
            coff = c * P
            pstart = pos0 + coff
            for h in range(2):
                j = (2 * cc + h) % 4
                j2 = (j + 2) % 4

                @pl.when(c > 0)
                def _():
                    pltpu.make_async_copy(
                        rbb[j2].at[0],
                        out_hbm.at[2 * h, pl.ds(pstart - P, P)],
                        ss[j2]).wait()
                    pltpu.make_async_copy(
                        rbb[j2].at[1],
                        out_hbm.at[2 * h + 1, pl.ds(pstart - P, P)],
                        ss[j2]).wait()

                @pl.when(c + 1 < N_CHUNKS)
                def _():
                    issue_gathers(c + 1, h, j2)

                pltpu.make_async_copy(
                    tbl_hbm.at[idx_all.at[2 * h, pl.ds(coff, P)]],
                    rbb[j].at[0], gs[j]).wait()
                pltpu.make_async_copy(
                    tbl_hbm.at[idx_all.at[2 * h + 1, pl.ds(coff, P)]],
                    rbb[j].at[1], gs[j]).wait()

                if h == 0:
                    pltpu.make_async_copy(
                        pe_hbm.at[pl.ds(pstart * HD, P * HD)], peb[cc],
                        qs[cc]).wait()

                aoff = pl.multiple_of(cp * 16, 16)
                for bl in range(2):
                    ivec = idx_all[2 * h + bl, pl.ds(aoff, 16)]
                    scale_v[pl.ds(bl * SSTR, 16)] = jnp.where(
                        ivec != PAD, 1.0, 0.0).astype(jnp.float32)

                def row_body(r, _):
                    sc0 = scale_v[pl.ds(8 * cc + r, 16)][0]
                    sc1 = scale_v[pl.ds(SSTR + 8 * cc + r, 16)][0]

                    @plsc.parallel_loop(0, D, 32, unroll=4)
                    def _add(off):
                        poff = pl.multiple_of(r * HD + off // 2, 16)
                        w = peb[cc][pl.ds(poff, 16)]
                        pa = lax.bitcast_convert_type(
                            w << 16, jnp.float32)
                        pb = lax.bitcast_convert_type(
                            w & jnp.int32(-65536), jnp.float32)
                        off2 = off + 16
                        rbb[j][0, r, pl.ds(off, 16)] = (
                            rbb[j][0, r, pl.ds(off, 16)] * sc0 + pa)
                        rbb[j][1, r, pl.ds(off, 16)] = (
                            rbb[j][1, r, pl.ds(off, 16)] * sc1 + pa)
                        rbb[j][0, r, pl.ds(off2, 16)] = (
                            rbb[j][0, r, pl.ds(off2, 16)] * sc0 + pb)
                        rbb[j][1, r, pl.ds(off2, 16)] = (
                            rbb[j][1, r, pl.ds(off2, 16)] * sc1 + pb)
                    return 0

                lax.fori_loop(0, P, row_body, 0)

                pltpu.async_copy(
                    rbb[j].at[0], out_hbm.at[2 * h, pl.ds(pstart, P)], ss[j])
                pltpu.async_copy(
                    rbb[j].at[1], out_hbm.at[2 * h + 1, pl.ds(pstart, P)],
                    ss[j])

                if h == 1:
                    @pl.when(c + 2 < N_CHUNKS)
                    def _():
                        pltpu.async_copy(
                            pe_hbm.at[pl.ds((pstart + 2 * P) * HD, P * HD)],
                            peb[cc], qs[cc])
        return 0

    lax.fori_loop(0, N_CHUNKS // 2, cpair_body, 0)
    last = pos0 + (N_CHUNKS - 1) * P
    pltpu.make_async_copy(rb2.at[0], out_hbm.at[0, pl.ds(last, P)], s2).wait()
    pltpu.make_async_copy(rb2.at[1], out_hbm.at[1, pl.ds(last, P)], s2).wait()
    pltpu.make_async_copy(rb3.at[0], out_hbm.at[2, pl.ds(last, P)], s3).wait()
    pltpu.make_async_copy(rb3.at[1], out_hbm.at[3, pl.ds(last, P)], s3).wait()


def kernel(x, token_emb_weight):
    pe = _pos_encoding(S, D)
    mesh = plsc.VectorSubcoreMesh(core_axis_name="c", subcore_axis_name="s")
    k = pl.kernel(
        _body,
        out_type=jax.ShapeDtypeStruct((B, S, D), jnp.float32),
        mesh=mesh,
        scratch_types=[
            pltpu.VMEM((B, POS_PER_W + 16), jnp.int32),
            pltpu.VMEM((2 * SSTR,), jnp.float32),
            pltpu.VMEM((P * D // 2,), jnp.int32),
            pltpu.VMEM((P * D // 2,), jnp.int32),
            pltpu.VMEM((2, P, D), jnp.float32),
            pltpu.VMEM((2, P, D), jnp.float32),
            pltpu.VMEM((2, P, D), jnp.float32),
            pltpu.VMEM((2, P, D), jnp.float32),
        ] + [pltpu.SemaphoreType.DMA] * 10,
    )
    return k(x, pe, token_emb_weight)
